# baseline (speedup 1.0000x reference)
import functools

import jax
import jax.numpy as jnp
from jax import lax
from jax.experimental import pallas as pl
from jax.experimental.pallas import tpu as pltpu
from jax.experimental.pallas import tpu_sc as plsc

FEATURES_DIM = 32
BATCH = 16384
N_FIELDS = 26
NUM_CORES = 2
NUM_SUBCORES = 16
NUM_WORKERS = NUM_CORES * NUM_SUBCORES
BB = 64
NBLK = BATCH // BB
BLK_PER_W = NBLK // NUM_WORKERS
CHUNK = BB * N_FIELDS
NBUF = 2

_mesh = plsc.VectorSubcoreMesh(core_axis_name="c", subcore_axis_name="s")

QTOT = 250000
QCHUNK = 64
RCHUNK = QCHUNK * 4
Q_PER_W = 7816
Q_LAST = QTOT - 31 * Q_PER_W


@functools.partial(
    pl.kernel,
    mesh=_mesh,
    out_type=jax.ShapeDtypeStruct((QTOT, 128), jnp.float32),
    scratch_types=[
        [pltpu.VMEM((RCHUNK, 32), jnp.float32)] * 2,
        [pltpu.VMEM((QCHUNK, 128), jnp.float32)] * 2,
        [pltpu.SemaphoreType.DMA] * 2,
        [pltpu.SemaphoreType.DMA] * 2,
    ],
    compiler_params=pltpu.CompilerParams(use_tc_tiling_on_sc=True),
)
def _depad_kernel(emb_hbm, out_hbm, in_v, pk_v, rsems, wsems):
    wid = lax.axis_index("s") * NUM_CORES + lax.axis_index("c")
    qbase = wid * Q_PER_W

    def repack(nrows, src, dst):
        for r in range(nrows):
            q, m = r // 4, r % 4
            dst[q, pl.ds(32 * m, 16)] = src[r, pl.ds(0, 16)]
            dst[q, pl.ds(32 * m + 16, 16)] = src[r, pl.ds(16, 16)]

    def do_chunk_pair(q0):
        h = [None, None]
        for b in range(2):
            h[b] = pltpu.async_copy(
                emb_hbm.at[pl.ds((q0 + b * QCHUNK) * 4, RCHUNK), :],
                in_v[b], rsems[b])
        w = [None, None]
        for b in range(2):
            h[b].wait()
            repack(RCHUNK, in_v[b], pk_v[b])
            w[b] = pltpu.async_copy(
                pk_v[b], out_hbm.at[pl.ds(q0 + b * QCHUNK, QCHUNK), :],
                wsems[b])
        for b in range(2):
            w[b].wait()

    npair = jnp.where(wid == 31, 60, 61)

    def pair_body(jj, carry):
        do_chunk_pair(qbase + jj * 2 * QCHUNK)
        return carry

    lax.fori_loop(0, npair, pair_body, 0)

    @pl.when(wid < 31)
    def _():
        q0 = qbase + 7808
        pltpu.sync_copy(emb_hbm.at[pl.ds(q0 * 4, 32), :],
                        in_v[0].at[pl.ds(0, 32), :])
        repack(32, in_v[0], pk_v[0])
        pltpu.sync_copy(pk_v[0].at[pl.ds(0, 8), :],
                        out_hbm.at[pl.ds(q0, 8), :])

    @pl.when(wid == 31)
    def _():
        q0 = qbase + 7680
        pltpu.sync_copy(emb_hbm.at[pl.ds(q0 * 4, 96), :],
                        in_v[0].at[pl.ds(0, 96), :])
        repack(96, in_v[0], pk_v[0])
        pltpu.sync_copy(pk_v[0].at[pl.ds(0, 24), :],
                        out_hbm.at[pl.ds(q0, 24), :])


@functools.partial(
    pl.kernel,
    mesh=_mesh,
    out_type=jax.ShapeDtypeStruct((BATCH, 32, 128), jnp.float32),
    scratch_types=[
        [pltpu.VMEM((CHUNK,), jnp.int32)] * NBUF,
        [pltpu.VMEM((CHUNK, FEATURES_DIM), jnp.float32)] * NBUF,
        [pltpu.SemaphoreType.DMA] * NBUF,
        [pltpu.SemaphoreType.DMA] * NBUF,
    ],
    compiler_params=pltpu.CompilerParams(use_tc_tiling_on_sc=False),
)
def _gather_kernel(idx_hbm, table_hbm, out_hbm, idx_vs, rows_v, gsems, wsems):
    wid = lax.axis_index("s") * NUM_CORES + lax.axis_index("c")
    base = wid * BLK_PER_W

    gh = [None] * BLK_PER_W

    def start_gather(j):
        b = j % NBUF
        pltpu.sync_copy(idx_hbm.at[pl.ds((base + j) * CHUNK, CHUNK)], idx_vs[b])
        gh[j] = pltpu.async_copy(
            table_hbm.at[idx_vs[b]], rows_v[b], gsems[b])

    for j in range(NBUF):
        start_gather(j)
    for j in range(BLK_PER_W):
        b = j % NBUF
        gh[j].wait()
        b0 = (base + j) * BB
        whs = [
            pltpu.async_copy(
                rows_v[b].at[pl.ds(bb * N_FIELDS, N_FIELDS), :],
                out_hbm.at[b0 + bb, pl.ds(0, N_FIELDS),
                           pl.ds(0, FEATURES_DIM)],
                wsems[b])
            for bb in range(BB)
        ]
        for w in whs:
            w.wait()
        nxt = j + NBUF
        if nxt < BLK_PER_W:
            start_gather(nxt)


def kernel(group_indices, embedding):
    idx = group_indices.reshape(-1).astype(jnp.int32)
    t_lin = _depad_kernel(embedding).reshape(1000000, 32)
    y = _gather_kernel(idx, t_lin)
    return y[:, :N_FIELDS, :FEATURES_DIM]


# speedup vs baseline: 1.2225x; 1.2225x over previous
import functools

import jax
import jax.numpy as jnp
from jax import lax
from jax.experimental import pallas as pl
from jax.experimental.pallas import tpu as pltpu
from jax.experimental.pallas import tpu_sc as plsc

FEATURES_DIM = 32
BATCH = 16384
N_FIELDS = 26
NUM_CORES = 2
NUM_SUBCORES = 16
NUM_WORKERS = NUM_CORES * NUM_SUBCORES
BB = 64
NBLK = BATCH // BB
BLK_PER_W = NBLK // NUM_WORKERS
CHUNK = BB * N_FIELDS
NBUF = 2

_mesh = plsc.VectorSubcoreMesh(core_axis_name="c", subcore_axis_name="s")

QTOT = 250000
QCHUNK = 64
RCHUNK = QCHUNK * 4
Q_PER_W = 7816
Q_LAST = QTOT - 31 * Q_PER_W


@functools.partial(
    pl.kernel,
    mesh=_mesh,
    out_type=jax.ShapeDtypeStruct((QTOT, 128), jnp.float32),
    scratch_types=[
        [pltpu.VMEM((RCHUNK // 8, 8, 32), jnp.float32)] * 2,
        [pltpu.VMEM((QCHUNK, 128), jnp.float32)] * 2,
        [pltpu.SemaphoreType.DMA] * 2,
        [pltpu.SemaphoreType.DMA] * 2,
    ],
    compiler_params=pltpu.CompilerParams(use_tc_tiling_on_sc=True),
)
def _depad_kernel(emb_hbm, out_hbm, in_v, pk_v, rsems, wsems):
    wid = lax.axis_index("s") * NUM_CORES + lax.axis_index("c")
    qbase = wid * Q_PER_W

    def repack(nrows, src, dst):
        for r in range(nrows):
            q, m = r // 4, r % 4
            dst[q, pl.ds(32 * m, 16)] = src[r // 8, r % 8, pl.ds(0, 16)]
            dst[q, pl.ds(32 * m + 16, 16)] = src[r // 8, r % 8, pl.ds(16, 16)]

    def do_chunk_pair(q0):
        h = [None, None]
        for b in range(2):
            h[b] = pltpu.async_copy(
                emb_hbm.at[pl.ds((q0 + b * QCHUNK) // 2, RCHUNK // 8), :, :],
                in_v[b], rsems[b])
        w = [None, None]
        for b in range(2):
            h[b].wait()
            repack(RCHUNK, in_v[b], pk_v[b])
            w[b] = pltpu.async_copy(
                pk_v[b], out_hbm.at[pl.ds(q0 + b * QCHUNK, QCHUNK), :],
                wsems[b])
        for b in range(2):
            w[b].wait()

    npair = jnp.where(wid == 31, 60, 61)

    def pair_body(jj, carry):
        do_chunk_pair(qbase + jj * 2 * QCHUNK)
        return carry

    lax.fori_loop(0, npair, pair_body, 0)

    @pl.when(wid < 31)
    def _():
        q0 = qbase + 7808
        pltpu.sync_copy(emb_hbm.at[pl.ds(q0 // 2, 4), :, :],
                        in_v[0].at[pl.ds(0, 4), :, :])
        repack(32, in_v[0], pk_v[0])
        pltpu.sync_copy(pk_v[0].at[pl.ds(0, 8), :],
                        out_hbm.at[pl.ds(q0, 8), :])

    @pl.when(wid == 31)
    def _():
        q0 = qbase + 7680
        pltpu.sync_copy(emb_hbm.at[pl.ds(q0 // 2, 12), :, :],
                        in_v[0].at[pl.ds(0, 12), :, :])
        repack(96, in_v[0], pk_v[0])
        pltpu.sync_copy(pk_v[0].at[pl.ds(0, 24), :],
                        out_hbm.at[pl.ds(q0, 24), :])


@functools.partial(
    pl.kernel,
    mesh=_mesh,
    out_type=jax.ShapeDtypeStruct((BATCH, 32, 128), jnp.float32),
    scratch_types=[
        [pltpu.VMEM((CHUNK,), jnp.int32)] * NBUF,
        [pltpu.VMEM((CHUNK, FEATURES_DIM), jnp.float32)] * NBUF,
        [pltpu.SemaphoreType.DMA] * NBUF,
        [pltpu.SemaphoreType.DMA] * NBUF,
    ],
    compiler_params=pltpu.CompilerParams(use_tc_tiling_on_sc=False),
)
def _gather_kernel(idx_hbm, table_hbm, out_hbm, idx_vs, rows_v, gsems, wsems):
    wid = lax.axis_index("s") * NUM_CORES + lax.axis_index("c")
    base = wid * BLK_PER_W

    gh = [None] * BLK_PER_W

    def start_gather(j):
        b = j % NBUF
        pltpu.sync_copy(idx_hbm.at[pl.ds((base + j) * CHUNK, CHUNK)], idx_vs[b])
        gh[j] = pltpu.async_copy(
            table_hbm.at[idx_vs[b]], rows_v[b], gsems[b])

    for j in range(NBUF):
        start_gather(j)
    for j in range(BLK_PER_W):
        b = j % NBUF
        gh[j].wait()
        b0 = (base + j) * BB
        whs = [
            pltpu.async_copy(
                rows_v[b].at[pl.ds(bb * N_FIELDS, N_FIELDS), :],
                out_hbm.at[b0 + bb, pl.ds(0, N_FIELDS),
                           pl.ds(0, FEATURES_DIM)],
                wsems[b])
            for bb in range(BB)
        ]
        for w in whs:
            w.wait()
        nxt = j + NBUF
        if nxt < BLK_PER_W:
            start_gather(nxt)


def kernel(group_indices, embedding):
    idx = group_indices.reshape(-1).astype(jnp.int32)
    t_lin = _depad_kernel(embedding.reshape(125000, 8, 32)).reshape(1000000, 32)
    y = _gather_kernel(idx, t_lin)
    return y[:, :N_FIELDS, :FEATURES_DIM]


# speedup vs baseline: 1.2312x; 1.0072x over previous
import functools

import jax
import jax.numpy as jnp
from jax import lax
from jax.experimental import pallas as pl
from jax.experimental.pallas import tpu as pltpu
from jax.experimental.pallas import tpu_sc as plsc

FEATURES_DIM = 32
BATCH = 16384
N_FIELDS = 26
NUM_CORES = 2
NUM_SUBCORES = 16
NUM_WORKERS = NUM_CORES * NUM_SUBCORES
BB = 64
NBLK = BATCH // BB
BLK_PER_W = NBLK // NUM_WORKERS
CHUNK = BB * N_FIELDS
NBUF = 2

_mesh = plsc.VectorSubcoreMesh(core_axis_name="c", subcore_axis_name="s")

QTOT = 250000
QCHUNK = 32
RCHUNK = QCHUNK * 4
NDB = 4
Q_PER_W = 7816
Q_LAST = QTOT - 31 * Q_PER_W


@functools.partial(
    pl.kernel,
    mesh=_mesh,
    out_type=jax.ShapeDtypeStruct((QTOT, 128), jnp.float32),
    scratch_types=[
        [pltpu.VMEM((RCHUNK // 8, 8, 32), jnp.float32)] * NDB,
        [pltpu.VMEM((QCHUNK, 128), jnp.float32)] * NDB,
        [pltpu.SemaphoreType.DMA] * NDB,
        [pltpu.SemaphoreType.DMA] * NDB,
    ],
    compiler_params=pltpu.CompilerParams(use_tc_tiling_on_sc=True),
)
def _depad_kernel(emb_hbm, out_hbm, in_v, pk_v, rsems, wsems):
    wid = lax.axis_index("s") * NUM_CORES + lax.axis_index("c")
    qbase = wid * Q_PER_W

    def repack(nrows, src, dst):
        for r in range(nrows):
            q, m = r // 4, r % 4
            dst[q, pl.ds(32 * m, 16)] = src[r // 8, r % 8, pl.ds(0, 16)]
            dst[q, pl.ds(32 * m + 16, 16)] = src[r // 8, r % 8, pl.ds(16, 16)]

    def do_chunk_quad(q0):
        h = [None] * NDB
        for b in range(NDB):
            h[b] = pltpu.async_copy(
                emb_hbm.at[pl.ds((q0 + b * QCHUNK) // 2, RCHUNK // 8), :, :],
                in_v[b], rsems[b])
        w = [None] * NDB
        for b in range(NDB):
            h[b].wait()
            repack(RCHUNK, in_v[b], pk_v[b])
            w[b] = pltpu.async_copy(
                pk_v[b], out_hbm.at[pl.ds(q0 + b * QCHUNK, QCHUNK), :],
                wsems[b])
        for b in range(NDB):
            w[b].wait()

    nquad = jnp.where(wid == 31, 60, 61)

    def quad_body(jj, carry):
        do_chunk_quad(qbase + jj * NDB * QCHUNK)
        return carry

    lax.fori_loop(0, nquad, quad_body, 0)

    @pl.when(wid < 31)
    def _():
        q0 = qbase + 7808
        pltpu.sync_copy(emb_hbm.at[pl.ds(q0 // 2, 4), :, :],
                        in_v[0].at[pl.ds(0, 4), :, :])
        repack(32, in_v[0], pk_v[0])
        pltpu.sync_copy(pk_v[0].at[pl.ds(0, 8), :],
                        out_hbm.at[pl.ds(q0, 8), :])

    @pl.when(wid == 31)
    def _():
        q0 = qbase + 7680
        pltpu.sync_copy(emb_hbm.at[pl.ds(q0 // 2, 12), :, :],
                        in_v[0].at[pl.ds(0, 12), :, :])
        repack(96, in_v[0], pk_v[0])
        pltpu.sync_copy(pk_v[0].at[pl.ds(0, 24), :],
                        out_hbm.at[pl.ds(q0, 24), :])


@functools.partial(
    pl.kernel,
    mesh=_mesh,
    out_type=jax.ShapeDtypeStruct((BATCH, 32, 128), jnp.float32),
    scratch_types=[
        [pltpu.VMEM((CHUNK,), jnp.int32)] * NBUF,
        [pltpu.VMEM((CHUNK, FEATURES_DIM), jnp.float32)] * NBUF,
        [pltpu.SemaphoreType.DMA] * NBUF,
        [pltpu.SemaphoreType.DMA] * NBUF,
    ],
    compiler_params=pltpu.CompilerParams(use_tc_tiling_on_sc=False),
)
def _gather_kernel(idx_hbm, table_hbm, out_hbm, idx_vs, rows_v, gsems, wsems):
    wid = lax.axis_index("s") * NUM_CORES + lax.axis_index("c")
    base = wid * BLK_PER_W

    gh = [None] * BLK_PER_W

    def start_gather(j):
        b = j % NBUF
        pltpu.sync_copy(idx_hbm.at[pl.ds((base + j) * CHUNK, CHUNK)], idx_vs[b])
        gh[j] = pltpu.async_copy(
            table_hbm.at[idx_vs[b]], rows_v[b], gsems[b])

    for j in range(NBUF):
        start_gather(j)
    for j in range(BLK_PER_W):
        b = j % NBUF
        gh[j].wait()
        b0 = (base + j) * BB
        whs = [
            pltpu.async_copy(
                rows_v[b].at[pl.ds(bb * N_FIELDS, N_FIELDS), :],
                out_hbm.at[b0 + bb, pl.ds(0, N_FIELDS),
                           pl.ds(0, FEATURES_DIM)],
                wsems[b])
            for bb in range(BB)
        ]
        for w in whs:
            w.wait()
        nxt = j + NBUF
        if nxt < BLK_PER_W:
            start_gather(nxt)


def kernel(group_indices, embedding):
    idx = group_indices.reshape(-1).astype(jnp.int32)
    t_lin = _depad_kernel(embedding.reshape(125000, 8, 32)).reshape(1000000, 32)
    y = _gather_kernel(idx, t_lin)
    return y[:, :N_FIELDS, :FEATURES_DIM]
